# initial kernel scaffold (unmeasured)
import jax
import jax.numpy as jnp
from jax import lax
from jax.experimental import pallas as pl
from jax.experimental.pallas import tpu as pltpu


def kernel(
    x,
):
    def body(*refs):
        pass

    out_shape = jax.ShapeDtypeStruct(..., jnp.float32)
    return pl.pallas_call(body, out_shape=out_shape)(...)


# baseline (device time: 196905 ns/iter reference)
import jax
import jax.numpy as jnp
from jax import lax
from jax.experimental import pallas as pl
from jax.experimental.pallas import tpu as pltpu


def kernel(x):
    m, n = x.shape

    def body(x_ref, out_ref, recv_buf, send_sem, recv_sem):
        my_x = lax.axis_index("x")
        my_y = lax.axis_index("y")
        my_z = lax.axis_index("z")
        nbr = (1 - my_x, my_y, my_z)

        barrier_sem = pltpu.get_barrier_semaphore()
        pl.semaphore_signal(
            barrier_sem, inc=1, device_id=nbr,
            device_id_type=pl.DeviceIdType.MESH,
        )
        pl.semaphore_wait(barrier_sem, 1)

        rdma = pltpu.make_async_remote_copy(
            src_ref=x_ref,
            dst_ref=recv_buf,
            send_sem=send_sem,
            recv_sem=recv_sem,
            device_id=nbr,
            device_id_type=pl.DeviceIdType.MESH,
        )
        rdma.start()
        rdma.wait()

        out_ref[...] = x_ref[...] + recv_buf[...]

    return pl.pallas_call(
        body,
        out_shape=jax.ShapeDtypeStruct((m, n), x.dtype),
        in_specs=[pl.BlockSpec(memory_space=pltpu.VMEM)],
        out_specs=pl.BlockSpec(memory_space=pltpu.VMEM),
        scratch_shapes=[
            pltpu.VMEM((m, n), x.dtype),
            pltpu.SemaphoreType.DMA,
            pltpu.SemaphoreType.DMA,
        ],
        compiler_params=pltpu.CompilerParams(collective_id=0),
    )(x)


# device time: 134181 ns/iter; 1.4675x vs baseline; 1.4675x over previous
import jax
import jax.numpy as jnp
from jax import lax
from jax.experimental import pallas as pl
from jax.experimental.pallas import tpu as pltpu


def kernel(x):
    m, n = x.shape
    Q = m // 4
    H = Q // 2

    def body(x_ref, out_ref, remote, send_sems, recv_sems):
        my_x = lax.axis_index("x")
        my_y = lax.axis_index("y")
        my_z = lax.axis_index("z")
        xn = (1 - my_x, my_y, my_z)
        yn = (my_x, 1 - my_y, my_z)
        zn = (my_x, my_y, 1 - my_z)

        q = 2 * my_y + my_z
        q_y = 2 * (1 - my_y) + my_z
        q_z = 2 * my_y + (1 - my_z)

        barrier_sem = pltpu.get_barrier_semaphore()
        for nbr in (xn, yn, zn):
            pl.semaphore_signal(
                barrier_sem, inc=1, device_id=nbr,
                device_id_type=pl.DeviceIdType.MESH,
            )
        pl.semaphore_wait(barrier_sem, 3)

        def copy(src, dst, sem_idx, dev):
            return pltpu.make_async_remote_copy(
                src_ref=src, dst_ref=dst,
                send_sem=send_sems.at[sem_idx],
                recv_sem=recv_sems.at[sem_idx],
                device_id=dev, device_id_type=pl.DeviceIdType.MESH,
            )

        p1 = copy(x_ref.at[pl.ds(q * Q, Q)], remote.at[pl.ds(q * Q, Q)], 0, xn)
        p1.start()
        p1.wait_recv()

        p2y = copy(remote.at[pl.ds(q * Q, Q)], remote.at[pl.ds(q * Q, Q)], 1, yn)
        p2z = copy(remote.at[pl.ds(q * Q, Q)], remote.at[pl.ds(q * Q, Q)], 2, zn)
        p2y.start()
        p2z.start()

        p2y.wait_recv()
        p3z = copy(
            remote.at[pl.ds(q_y * Q, H)], remote.at[pl.ds(q_y * Q, H)], 3, zn
        )
        p3z.start()
        p2z.wait_recv()
        p3y = copy(
            remote.at[pl.ds(q_z * Q + H, H)],
            remote.at[pl.ds(q_z * Q + H, H)], 4, yn,
        )
        p3y.start()

        p3z.wait_recv()
        p3y.wait_recv()

        p1.wait_send()
        p2y.wait_send()
        p2z.wait_send()
        p3z.wait_send()
        p3y.wait_send()

        out_ref[...] = x_ref[...] + remote[...]

    return pl.pallas_call(
        body,
        out_shape=jax.ShapeDtypeStruct((m, n), x.dtype),
        in_specs=[pl.BlockSpec(memory_space=pltpu.VMEM)],
        out_specs=pl.BlockSpec(memory_space=pltpu.VMEM),
        scratch_shapes=[
            pltpu.VMEM((m, n), x.dtype),
            pltpu.SemaphoreType.DMA((5,)),
            pltpu.SemaphoreType.DMA((5,)),
        ],
        compiler_params=pltpu.CompilerParams(collective_id=0),
    )(x)


# device time: 91965 ns/iter; 2.1411x vs baseline; 1.4590x over previous
import jax
import jax.numpy as jnp
from jax import lax
from jax.experimental import pallas as pl
from jax.experimental.pallas import tpu as pltpu

NQ = 4

DX = 344
RZ = 336
RY = 344


def kernel(x):
    m, n = x.shape
    Q = m // 4
    C = Q // NQ

    def body(
        x_ref, out_ref, remote,
        p1_ssem, p1_rsem, p2y_ssem, p2y_rsem, p2z_ssem, p2z_rsem,
        rl_ssem, rl_rsem,
    ):
        my_x = lax.axis_index("x")
        my_y = lax.axis_index("y")
        my_z = lax.axis_index("z")
        xn = (1 - my_x, my_y, my_z)
        yn = (my_x, 1 - my_y, my_z)
        zn = (my_x, my_y, 1 - my_z)

        q = 2 * my_y + my_z
        q_y = 2 * (1 - my_y) + my_z
        q_z = 2 * my_y + (1 - my_z)
        q_d = 2 * (1 - my_y) + (1 - my_z)

        barrier_sem = pltpu.get_barrier_semaphore()
        for nbr in (xn, yn, zn):
            pl.semaphore_signal(
                barrier_sem, inc=1, device_id=nbr,
                device_id_type=pl.DeviceIdType.MESH,
            )
        pl.semaphore_wait(barrier_sem, 3)

        def copy(src, dst, ssem, rsem, dev):
            return pltpu.make_async_remote_copy(
                src_ref=src, dst_ref=dst, send_sem=ssem, recv_sem=rsem,
                device_id=dev, device_id_type=pl.DeviceIdType.MESH,
            )

        p1 = [
            copy(
                x_ref.at[pl.ds(q * Q + c * C, C)],
                remote.at[pl.ds(q * Q + c * C, C)],
                p1_ssem.at[c], p1_rsem.at[c], xn,
            )
            for c in range(NQ)
        ]
        p1d = copy(
            x_ref.at[pl.ds(q_d * Q, DX)],
            remote.at[pl.ds(q_d * Q, DX)],
            p1_ssem.at[NQ], p1_rsem.at[NQ], xn,
        )
        for r in p1:
            r.start()
        p1d.start()

        p2y_in = [
            copy(
                remote.at[pl.ds(q_y * Q + c * C, C)],
                remote.at[pl.ds(q_y * Q + c * C, C)],
                p2y_ssem.at[c], p2y_rsem.at[c], yn,
            )
            for c in range(NQ)
        ]
        p2z_in = [
            copy(
                remote.at[pl.ds(q_z * Q + c * C, C)],
                remote.at[pl.ds(q_z * Q + c * C, C)],
                p2z_ssem.at[c], p2z_rsem.at[c], zn,
            )
            for c in range(NQ)
        ]

        p2y_out, p2z_out = [], []
        for c in range(NQ):
            p1[c].wait_recv()
            src = remote.at[pl.ds(q * Q + c * C, C)]
            ry = copy(src, src, p2y_ssem.at[c], p2y_rsem.at[c], yn)
            rz = copy(src, src, p2z_ssem.at[c], p2z_rsem.at[c], zn)
            ry.start()
            rz.start()
            p2y_out.append(ry)
            p2z_out.append(rz)

        for c in range(3):
            p2y_in[c].wait_recv()
        rlz_out = copy(
            remote.at[pl.ds(q_y * Q + DX, RZ)],
            remote.at[pl.ds(q_y * Q + DX, RZ)],
            rl_ssem.at[0], rl_rsem.at[0], zn,
        )
        rlz_out.start()

        for c in range(NQ):
            p2z_in[c].wait_recv()
        rly_out = copy(
            remote.at[pl.ds(q_z * Q + DX + RZ, RY)],
            remote.at[pl.ds(q_z * Q + DX + RZ, RY)],
            rl_ssem.at[1], rl_rsem.at[1], yn,
        )
        rly_out.start()

        out_ref[pl.ds(q * Q, Q), :] = (
            x_ref[pl.ds(q * Q, Q), :] + remote[pl.ds(q * Q, Q), :]
        )
        out_ref[pl.ds(q_z * Q, Q), :] = (
            x_ref[pl.ds(q_z * Q, Q), :] + remote[pl.ds(q_z * Q, Q), :]
        )
        p2y_in[3].wait_recv()
        out_ref[pl.ds(q_y * Q, Q), :] = (
            x_ref[pl.ds(q_y * Q, Q), :] + remote[pl.ds(q_y * Q, Q), :]
        )

        rlz_in = copy(
            remote.at[pl.ds(q_d * Q + DX, RZ)],
            remote.at[pl.ds(q_d * Q + DX, RZ)],
            rl_ssem.at[0], rl_rsem.at[0], zn,
        )
        rly_in = copy(
            remote.at[pl.ds(q_d * Q + DX + RZ, RY)],
            remote.at[pl.ds(q_d * Q + DX + RZ, RY)],
            rl_ssem.at[1], rl_rsem.at[1], yn,
        )
        p1d.wait_recv()
        rlz_in.wait_recv()
        rly_in.wait_recv()
        out_ref[pl.ds(q_d * Q, Q), :] = (
            x_ref[pl.ds(q_d * Q, Q), :] + remote[pl.ds(q_d * Q, Q), :]
        )

        for r in p1:
            r.wait_send()
        p1d.wait_send()
        for r in p2y_out:
            r.wait_send()
        for r in p2z_out:
            r.wait_send()
        rlz_out.wait_send()
        rly_out.wait_send()

    return pl.pallas_call(
        body,
        out_shape=jax.ShapeDtypeStruct((m, n), x.dtype),
        in_specs=[pl.BlockSpec(memory_space=pltpu.VMEM)],
        out_specs=pl.BlockSpec(memory_space=pltpu.VMEM),
        scratch_shapes=[
            pltpu.VMEM((m, n), x.dtype),
            pltpu.SemaphoreType.DMA((NQ + 1,)),
            pltpu.SemaphoreType.DMA((NQ + 1,)),
            pltpu.SemaphoreType.DMA((NQ,)),
            pltpu.SemaphoreType.DMA((NQ,)),
            pltpu.SemaphoreType.DMA((NQ,)),
            pltpu.SemaphoreType.DMA((NQ,)),
            pltpu.SemaphoreType.DMA((2,)),
            pltpu.SemaphoreType.DMA((2,)),
        ],
        compiler_params=pltpu.CompilerParams(collective_id=0),
    )(x)


# device time: 86285 ns/iter; 2.2820x vs baseline; 1.0658x over previous
import jax
import jax.numpy as jnp
from jax import lax
from jax.experimental import pallas as pl
from jax.experimental.pallas import tpu as pltpu

NQ = 8

DX = 344
RZ = 336
RY = 344


def kernel(x):
    m, n = x.shape
    Q = m // 4
    C = Q // NQ
    n_rlz = -(-(DX + RZ) // C)

    def body(
        x_ref, out_ref, remote,
        p1_ssem, p1_rsem, p2y_ssem, p2y_rsem, p2z_ssem, p2z_rsem,
        rl_ssem, rl_rsem,
    ):
        my_x = lax.axis_index("x")
        my_y = lax.axis_index("y")
        my_z = lax.axis_index("z")
        xn = (1 - my_x, my_y, my_z)
        yn = (my_x, 1 - my_y, my_z)
        zn = (my_x, my_y, 1 - my_z)

        q = 2 * my_y + my_z
        q_y = 2 * (1 - my_y) + my_z
        q_z = 2 * my_y + (1 - my_z)
        q_d = 2 * (1 - my_y) + (1 - my_z)

        barrier_sem = pltpu.get_barrier_semaphore()
        for nbr in (xn, yn, zn):
            pl.semaphore_signal(
                barrier_sem, inc=1, device_id=nbr,
                device_id_type=pl.DeviceIdType.MESH,
            )
        pl.semaphore_wait(barrier_sem, 3)

        def copy(src, dst, ssem, rsem, dev):
            return pltpu.make_async_remote_copy(
                src_ref=src, dst_ref=dst, send_sem=ssem, recv_sem=rsem,
                device_id=dev, device_id_type=pl.DeviceIdType.MESH,
            )

        def add_rows(start, size):
            out_ref[pl.ds(start, size), :] = (
                x_ref[pl.ds(start, size), :] + remote[pl.ds(start, size), :]
            )

        p1 = [
            copy(
                x_ref.at[pl.ds(q * Q + c * C, C)],
                remote.at[pl.ds(q * Q + c * C, C)],
                p1_ssem.at[c], p1_rsem.at[c], xn,
            )
            for c in range(NQ)
        ]
        p1d = copy(
            x_ref.at[pl.ds(q_d * Q, DX)],
            remote.at[pl.ds(q_d * Q, DX)],
            p1_ssem.at[NQ], p1_rsem.at[NQ], xn,
        )
        for r in p1:
            r.start()
        p1d.start()

        p2y_in = [
            copy(
                remote.at[pl.ds(q_y * Q + c * C, C)],
                remote.at[pl.ds(q_y * Q + c * C, C)],
                p2y_ssem.at[c], p2y_rsem.at[c], yn,
            )
            for c in range(NQ)
        ]
        p2z_in = [
            copy(
                remote.at[pl.ds(q_z * Q + c * C, C)],
                remote.at[pl.ds(q_z * Q + c * C, C)],
                p2z_ssem.at[c], p2z_rsem.at[c], zn,
            )
            for c in range(NQ)
        ]

        p2y_out, p2z_out = [], []
        for c in range(NQ):
            p1[c].wait_recv()
            src = remote.at[pl.ds(q * Q + c * C, C)]
            ry = copy(src, src, p2y_ssem.at[c], p2y_rsem.at[c], yn)
            rz = copy(src, src, p2z_ssem.at[c], p2z_rsem.at[c], zn)
            ry.start()
            rz.start()
            p2y_out.append(ry)
            p2z_out.append(rz)
            add_rows(q * Q + c * C, C)

        for c in range(n_rlz):
            p2y_in[c].wait_recv()
        rlz_out = copy(
            remote.at[pl.ds(q_y * Q + DX, RZ)],
            remote.at[pl.ds(q_y * Q + DX, RZ)],
            rl_ssem.at[0], rl_rsem.at[0], zn,
        )
        rlz_out.start()

        for c in range(NQ):
            p2z_in[c].wait_recv()
        rly_out = copy(
            remote.at[pl.ds(q_z * Q + DX + RZ, RY)],
            remote.at[pl.ds(q_z * Q + DX + RZ, RY)],
            rl_ssem.at[1], rl_rsem.at[1], yn,
        )
        rly_out.start()

        add_rows(q_z * Q, Q)
        for c in range(n_rlz, NQ):
            p2y_in[c].wait_recv()
        add_rows(q_y * Q, Q)

        rlz_in = copy(
            remote.at[pl.ds(q_d * Q + DX, RZ)],
            remote.at[pl.ds(q_d * Q + DX, RZ)],
            rl_ssem.at[0], rl_rsem.at[0], zn,
        )
        rly_in = copy(
            remote.at[pl.ds(q_d * Q + DX + RZ, RY)],
            remote.at[pl.ds(q_d * Q + DX + RZ, RY)],
            rl_ssem.at[1], rl_rsem.at[1], yn,
        )
        p1d.wait_recv()
        rlz_in.wait_recv()
        add_rows(q_d * Q, DX + RZ)
        rly_in.wait_recv()
        add_rows(q_d * Q + DX + RZ, RY)

        for r in p1:
            r.wait_send()
        p1d.wait_send()
        for r in p2y_out:
            r.wait_send()
        for r in p2z_out:
            r.wait_send()
        rlz_out.wait_send()
        rly_out.wait_send()

    return pl.pallas_call(
        body,
        out_shape=jax.ShapeDtypeStruct((m, n), x.dtype),
        in_specs=[pl.BlockSpec(memory_space=pltpu.VMEM)],
        out_specs=pl.BlockSpec(memory_space=pltpu.VMEM),
        scratch_shapes=[
            pltpu.VMEM((m, n), x.dtype),
            pltpu.SemaphoreType.DMA((NQ + 1,)),
            pltpu.SemaphoreType.DMA((NQ + 1,)),
            pltpu.SemaphoreType.DMA((NQ,)),
            pltpu.SemaphoreType.DMA((NQ,)),
            pltpu.SemaphoreType.DMA((NQ,)),
            pltpu.SemaphoreType.DMA((NQ,)),
            pltpu.SemaphoreType.DMA((2,)),
            pltpu.SemaphoreType.DMA((2,)),
        ],
        compiler_params=pltpu.CompilerParams(collective_id=0),
    )(x)
